# baseline (speedup 1.0000x reference)
import functools
import math

import jax
import jax.numpy as jnp
from jax import lax
from jax.experimental import pallas as pl
from jax.experimental.pallas import tpu as pltpu
from jax.experimental.pallas import tpu_sc as plsc

F32 = jnp.float32
NC = 2
NS = 16
NW = NC * NS


_GDN = lax.GatherDimensionNumbers(
    offset_dims=(), collapsed_slice_dims=(0,), start_index_map=(0,))


def _shfl(v, idx):
    return lax.gather(v, idx[:, None], _GDN, slice_sizes=(1,),
                      mode=lax.GatherScatterMode.PROMISE_IN_BOUNDS)


def _hsum_splat(v, lane):
    for sh in (8, 4, 2, 1):
        v = v + _shfl(v, lane ^ sh)
    return v


def _proj_body(x_ref, wl_ref, wr_ref, xl_ref, xr_ref):
    xb = x_ref[...]
    xl_ref[...] = jnp.dot(xb, wl_ref[...], preferred_element_type=F32)
    xr_ref[...] = jnp.dot(xb, wr_ref[...], preferred_element_type=F32)


def _proj(x, Wl, Wr, blk):
    n, d = x.shape
    grid = n // blk
    return pl.pallas_call(
        _proj_body,
        grid=(grid,),
        in_specs=[
            pl.BlockSpec((blk, d), lambda i: (i, 0)),
            pl.BlockSpec((d, d), lambda i: (0, 0)),
            pl.BlockSpec((d, d), lambda i: (0, 0)),
        ],
        out_specs=[
            pl.BlockSpec((blk, d), lambda i: (i, 0)),
            pl.BlockSpec((blk, d), lambda i: (i, 0)),
        ],
        out_shape=[
            jax.ShapeDtypeStruct((n, d), F32),
            jax.ShapeDtypeStruct((n, d), F32),
        ],
    )(x, Wl, Wr)


def _edge_pass_body(NB, B, NP, H, C,
                    xl_hbm, xr_hbm, src_hbm, dst_hbm, att_hbm, zu_hbm, zd_hbm,
                    u_out, den_out,
                    att_v, didx2d, sb0, sb1,
                    xlr0, xlr1, xrr0, xrr1, msg, exb,
                    u_acc, den_acc,
                    isem0, isem1, gsl0, gsl1, gsr0, gsr1, ssu, ssd):
    sbuf = (sb0, sb1)
    xlrs = (xlr0, xlr1)
    xrrs = (xrr0, xrr1)
    isem = (isem0, isem1)
    gsl = (gsl0, gsl1)
    gsr = (gsr0, gsr1)
    cid = lax.axis_index("c")
    sid = lax.axis_index("s")
    wid = sid * NC + cid
    D = H * C
    KV = D // 16
    EPW = NB * B

    pltpu.sync_copy(att_hbm, att_v)

    row0 = jnp.where(sid < 4, sid * 632, 2528 + (sid - 4) * 624)

    @pl.when(sid < 4)
    def _():
        pltpu.sync_copy(zu_hbm.at[pl.ds(row0, 632)],
                        u_acc.at[pl.ds(row0, 632)])
        pltpu.sync_copy(zd_hbm.at[pl.ds(row0, 632)],
                        den_acc.at[pl.ds(row0, 632)])

    @pl.when(sid >= 4)
    def _():
        pltpu.sync_copy(zu_hbm.at[pl.ds(row0, 624)],
                        u_acc.at[pl.ds(row0, 624)])
        pltpu.sync_copy(zd_hbm.at[pl.ds(row0, 624)],
                        den_acc.at[pl.ds(row0, 624)])

    plsc.subcore_barrier()

    attv = [att_v[pl.ds(16 * k, 16)] for k in range(KV)]
    lane = lax.broadcasted_iota(jnp.int32, (16,), 0)
    base = wid * EPW

    pltpu.sync_copy(dst_hbm.at[wid], didx2d)

    def compute(b):
        xlr, xrr = xlrs[b], xrrs[b]

        @plsc.parallel_loop(0, B, unroll=2)
        def _edge(e):
            a = [xlr[e, pl.ds(16 * k, 16)] for k in range(KV)]
            bb = [xrr[e, pl.ds(16 * k, 16)] for k in range(KV)]
            p = []
            for k in range(KV):
                sv = a[k] + bb[k]
                tv = jnp.maximum(sv, 0.2 * sv)
                p.append(tv * attv[k])
            exs = []
            for h in range(H):
                q = p[2 * h] + p[2 * h + 1]
                ev = jnp.exp(_hsum_splat(q, lane))
                exs.append(ev)
                msg[e, pl.ds(32 * h, 16)] = a[2 * h] * ev
                msg[e, pl.ds(32 * h + 16, 16)] = a[2 * h + 1] * ev
            exrow = jnp.where(
                lane == 0, exs[0],
                jnp.where(lane == 1, exs[1],
                          jnp.where(lane == 2, exs[2], exs[3])))
            exb[e, :] = exrow

    pltpu.sync_copy(src_hbm.at[pl.ds(base, B)], sbuf[0])
    pltpu.async_copy(xl_hbm.at[sbuf[0]], xlrs[0], gsl[0])
    pltpu.async_copy(xr_hbm.at[didx2d.at[0]], xrrs[0], gsr[0])
    pltpu.async_copy(src_hbm.at[pl.ds(base + B, B)], sbuf[1], isem[1])

    @pl.loop(0, NB, step=2)
    def _batch2(g0):
        for b in range(2):
            g = g0 + b
            bn = 1 - b
            pltpu.make_async_copy(xl_hbm.at[sbuf[b]], xlrs[b],
                                  gsl[b]).wait()
            pltpu.make_async_copy(xr_hbm.at[didx2d.at[g]], xrrs[b],
                                  gsr[b]).wait()

            @pl.when(g + 2 < NB)
            def _():
                pltpu.async_copy(
                    src_hbm.at[pl.ds(base + (g + 2) * B, B)],
                    sbuf[b], isem[b])

            @pl.when(g + 1 < NB)
            def _():
                pltpu.make_async_copy(
                    src_hbm.at[pl.ds(base + (g + 1) * B, B)],
                    sbuf[bn], isem[bn]).wait()
                pltpu.async_copy(xl_hbm.at[sbuf[bn]], xlrs[bn], gsl[bn])
                pltpu.async_copy(xr_hbm.at[didx2d.at[g + 1]], xrrs[bn],
                                 gsr[bn])

            @pl.when(g >= 1)
            def _():
                pltpu.make_async_copy(msg, u_acc.at[didx2d.at[g]],
                                      ssu).wait()
                pltpu.make_async_copy(exb, den_acc.at[didx2d.at[g]],
                                      ssd).wait()

            compute(b)
            pltpu.async_copy(msg, u_acc.at[didx2d.at[g]], ssu, add=True)
            pltpu.async_copy(exb, den_acc.at[didx2d.at[g]], ssd, add=True)

    pltpu.make_async_copy(msg, u_acc.at[didx2d.at[NB - 1]], ssu).wait()
    pltpu.make_async_copy(exb, den_acc.at[didx2d.at[NB - 1]], ssd).wait()

    plsc.subcore_barrier()

    @pl.when(sid < 4)
    def _():
        pltpu.sync_copy(u_acc.at[pl.ds(row0, 632)],
                        u_out.at[cid, pl.ds(row0, 632)])
        pltpu.sync_copy(den_acc.at[pl.ds(row0, 632)],
                        den_out.at[cid, pl.ds(row0, 632)])

    @pl.when(sid >= 4)
    def _():
        pltpu.sync_copy(u_acc.at[pl.ds(row0, 624)],
                        u_out.at[cid, pl.ds(row0, 624)])
        pltpu.sync_copy(den_acc.at[pl.ds(row0, 624)],
                        den_out.at[cid, pl.ds(row0, 624)])


def _edge_pass(xl, xr, src, dst, att_flat, H, C, NP):
    N, D = xl.shape
    E = src.shape[0]
    EPW = E // NW
    B = 40
    NB = EPW // B
    mesh = plsc.VectorSubcoreMesh(core_axis_name="c", subcore_axis_name="s")
    body = functools.partial(_edge_pass_body, NB, B, NP, H, C)
    f = pl.kernel(
        body,
        out_type=[
            jax.ShapeDtypeStruct((NC, NP, D), F32),
            jax.ShapeDtypeStruct((NC, NP, 16), F32),
        ],
        mesh=mesh,
        compiler_params=pltpu.CompilerParams(use_tc_tiling_on_sc=False),
        scratch_types=[
            pltpu.VMEM((D,), F32),
            pltpu.VMEM((NB, B), jnp.int32),
            pltpu.VMEM((B,), jnp.int32),
            pltpu.VMEM((B,), jnp.int32),
            pltpu.VMEM((B, D), F32),
            pltpu.VMEM((B, D), F32),
            pltpu.VMEM((B, D), F32),
            pltpu.VMEM((B, D), F32),
            pltpu.VMEM((B, D), F32),
            pltpu.VMEM((B, 16), F32),
            pltpu.VMEM_SHARED((NP, D), F32),
            pltpu.VMEM_SHARED((NP, 16), F32),
        ] + [pltpu.SemaphoreType.DMA] * 8,
    )
    zu = jnp.zeros((NP, D), F32)
    zd = jnp.zeros((NP, 16), F32)
    return f(xl, xr, src, dst.reshape(NW, NB, B), att_flat, zu, zd)


def _gin_agg_body(NB, B, NP,
                  x1_hbm, src_hbm, dst_hbm, zu_hbm, agg_out,
                  didx2d, sb0, sb1, sb2, sb3, r0_, r1_, r2_, r3_, acc,
                  is0, is1, is2, is3, gs0, gs1, gs2, gs3,
                  ss0, ss1, ss2, ss3):
    cid = lax.axis_index("c")
    sid = lax.axis_index("s")
    wid = sid * NC + cid
    sbuf = (sb0, sb1, sb2, sb3)
    rows = (r0_, r1_, r2_, r3_)
    isem = (is0, is1, is2, is3)
    gs = (gs0, gs1, gs2, gs3)
    ss = (ss0, ss1, ss2, ss3)
    rows_per_tile = NP // NS

    row0 = sid * rows_per_tile
    pltpu.sync_copy(zu_hbm.at[pl.ds(row0, rows_per_tile)],
                    acc.at[pl.ds(row0, rows_per_tile)])
    plsc.subcore_barrier()

    pltpu.sync_copy(dst_hbm.at[wid], didx2d)

    pltpu.sync_copy(src_hbm.at[wid, 0], sbuf[0])
    pltpu.async_copy(x1_hbm.at[sbuf[0]], rows[0], gs[0])
    pltpu.async_copy(src_hbm.at[wid, 1], sbuf[1], isem[1])

    @pl.loop(0, NB, step=4)
    def _batch4(g0):
        for b in range(4):
            g = g0 + b
            b1 = (b + 1) % 4
            b2 = (b + 2) % 4
            pltpu.make_async_copy(x1_hbm.at[sbuf[b]], rows[b],
                                  gs[b]).wait()

            @pl.when(g + 2 < NB)
            def _():
                pltpu.async_copy(src_hbm.at[wid, g + 2], sbuf[b2],
                                 isem[b2])

            @pl.when(g + 1 < NB)
            def _():
                pltpu.make_async_copy(src_hbm.at[wid, g + 1],
                                      sbuf[b1], isem[b1]).wait()

                @pl.when(g >= 3)
                def _():
                    pltpu.make_async_copy(rows[b1],
                                          acc.at[didx2d.at[g]],
                                          ss[b1]).wait()

                pltpu.async_copy(x1_hbm.at[sbuf[b1]], rows[b1], gs[b1])

            pltpu.async_copy(rows[b], acc.at[didx2d.at[g]], ss[b],
                             add=True)

    for b in (0, 1, 2, 3):
        pltpu.make_async_copy(rows[b], acc.at[didx2d.at[b]], ss[b]).wait()

    plsc.subcore_barrier()
    pltpu.sync_copy(acc.at[pl.ds(row0, rows_per_tile)],
                    agg_out.at[cid, pl.ds(row0, rows_per_tile)])


def _gin_agg(x1, src, dst, NP):
    N, D = x1.shape
    E = src.shape[0]
    EPW = E // NW
    B = 50
    NB = EPW // B
    mesh = plsc.VectorSubcoreMesh(core_axis_name="c", subcore_axis_name="s")
    body = functools.partial(_gin_agg_body, NB, B, NP)
    f = pl.kernel(
        body,
        out_type=jax.ShapeDtypeStruct((NC, NP, D), F32),
        mesh=mesh,
        compiler_params=pltpu.CompilerParams(use_tc_tiling_on_sc=False),
        scratch_types=[
            pltpu.VMEM((NB, B), jnp.int32),
            pltpu.VMEM((B,), jnp.int32),
            pltpu.VMEM((B,), jnp.int32),
            pltpu.VMEM((B,), jnp.int32),
            pltpu.VMEM((B,), jnp.int32),
            pltpu.VMEM((B, D), F32),
            pltpu.VMEM((B, D), F32),
            pltpu.VMEM((B, D), F32),
            pltpu.VMEM((B, D), F32),
            pltpu.VMEM_SHARED((NP, D), F32),
        ] + [pltpu.SemaphoreType.DMA] * 12,
    )
    zu = jnp.zeros((NP, D), F32)
    return f(x1, src.reshape(NW, NB, B), dst.reshape(NW, NB, B), zu)


def _finalize_body(xl_ref, xr_ref, u0_ref, u1_ref, d0_ref, d1_ref,
                   att_ref, mh_ref, md_ref, bias_ref, x1_ref):
    xl = xl_ref[...]
    blk, d = xl.shape
    s = xl + xr_ref[...]
    t = jnp.maximum(s, 0.2 * s) * att_ref[...]
    sfull = jnp.dot(t, mh_ref[...], preferred_element_type=F32)
    exs = jnp.exp(sfull)
    den = (d0_ref[...] + d1_ref[...]).reshape(blk, 16)
    denf = jnp.dot(den, md_ref[...], preferred_element_type=F32)
    num = (u0_ref[...] + u1_ref[...]).reshape(blk, d) + exs * xl
    gat = num / (denf + exs + 1e-16) + bias_ref[...]
    x1_ref[...] = jnp.maximum(gat, 0.01 * gat)


def _finalize(xl, xr, u, den, att_row, mhead, mden, bias_row, blk):
    n, d = xl.shape
    grid = n // blk
    nspec = pl.BlockSpec((blk, d), lambda i: (i, 0))
    wspec0 = pl.BlockSpec((1, d), lambda i: (0, 0))
    return pl.pallas_call(
        _finalize_body,
        grid=(grid,),
        in_specs=[
            nspec, nspec,
            pl.BlockSpec((1, blk, d), lambda i: (0, i, 0)),
            pl.BlockSpec((1, blk, d), lambda i: (1, i, 0)),
            pl.BlockSpec((1, blk, 16), lambda i: (0, i, 0)),
            pl.BlockSpec((1, blk, 16), lambda i: (1, i, 0)),
            wspec0,
            pl.BlockSpec((d, d), lambda i: (0, 0)),
            pl.BlockSpec((16, d), lambda i: (0, 0)),
            wspec0,
        ],
        out_specs=nspec,
        out_shape=jax.ShapeDtypeStruct((n, d), F32),
    )(xl, xr, u, u, den, den, att_row, mhead, mden, bias_row)


def _mlp_pool_body(G, x1_ref, a0_ref, a1_ref, bt_ref, w1_ref, b1_ref,
                   w2_ref, b2_ref, out_ref):
    i = pl.program_id(0)
    x1 = x1_ref[...]
    h0 = x1 + (a0_ref[...] + a1_ref[...]).reshape(x1.shape)
    h1 = jnp.maximum(jnp.dot(h0, w1_ref[...], preferred_element_type=F32)
                     + b1_ref[...], 0.0)
    h2 = jnp.maximum(jnp.dot(h1, w2_ref[...], preferred_element_type=F32)
                     + b2_ref[...], 0.0)
    blk = h2.shape[0]
    gids = lax.broadcasted_iota(jnp.int32, (G, blk), 0)
    bt = bt_ref[...].reshape(1, blk)
    onehot = jnp.where(gids == jnp.broadcast_to(bt, (G, blk)),
                       1.0, 0.0).astype(F32)
    pp = jnp.dot(onehot, h2, preferred_element_type=F32)

    @pl.when(i == 0)
    def _():
        out_ref[...] = jnp.zeros_like(out_ref)

    out_ref[...] += pp


def _mlp_pool(x1, agg, bt, w1, b1_row, w2, b2_row, G, blk):
    n, d = x1.shape
    grid = n // blk
    nspec = pl.BlockSpec((blk, d), lambda i: (i, 0))
    wspec = pl.BlockSpec((d, d), lambda i: (0, 0))
    bspec = pl.BlockSpec((1, d), lambda i: (0, 0))
    return pl.pallas_call(
        functools.partial(_mlp_pool_body, G),
        grid=(grid,),
        in_specs=[
            nspec,
            pl.BlockSpec((1, blk, d), lambda i: (0, i, 0)),
            pl.BlockSpec((1, blk, d), lambda i: (1, i, 0)),
            pl.BlockSpec((1, 1, blk), lambda i: (i, 0, 0)),
            wspec, bspec, wspec, bspec,
        ],
        out_specs=pl.BlockSpec((G, d), lambda i: (0, 0)),
        out_shape=jax.ShapeDtypeStruct((G, d), F32),
    )(x1, agg, agg, bt, w1, b1_row, w2, b2_row)


@jax.jit
def _run(x, edge_index, batch, Wl, Wr, att, gat_bias, gin_w1, gin_b1,
         bn_gamma, bn_beta, gin_w2, gin_b2):
    N, D = x.shape
    H, C = att.shape
    G = 64
    blk = 1000

    src = edge_index[0]
    dst = edge_index[1]

    att_flat = att.reshape(D)
    att_row = att_flat.reshape(1, D)
    ci = jnp.arange(D, dtype=jnp.int32) // C
    mhead = (ci[:, None] == ci[None, :]).astype(F32)
    mden = (jnp.arange(16, dtype=jnp.int32)[:, None] == ci[None, :]).astype(F32)
    bias_row = gat_bias.reshape(1, D)
    bn_s = bn_gamma / math.sqrt(1.0 + 1e-5)
    w1p = gin_w1 * bn_s[None, :]
    b1_row = (gin_b1 * bn_s + bn_beta).reshape(1, D)
    b2_row = gin_b2.reshape(1, D)
    bt = batch.reshape(N // blk, 1, blk)

    NPE = 10016
    NP = 10240
    xl, xr = _proj(x, Wl, Wr, blk)
    u, den = _edge_pass(xl, xr, src, dst, att_flat, H, C, NPE)
    x1 = _finalize(xl, xr, u, den, att_row, mhead, mden, bias_row, blk)
    agg = _gin_agg(x1, src, dst, NP)
    return _mlp_pool(x1, agg, bt, w1p, b1_row, gin_w2, b2_row, G, blk)


def kernel(x, edge_index, batch, Wl, Wr, att, gat_bias, gin_w1, gin_b1,
           bn_gamma, bn_beta, gin_w2, gin_b2):
    return _run(x, edge_index, batch, Wl, Wr, att, gat_bias, gin_w1, gin_b1,
                bn_gamma, bn_beta, gin_w2, gin_b2)


# speedup vs baseline: 1.1536x; 1.1536x over previous
import functools
import math

import jax
import jax.numpy as jnp
from jax import lax
from jax.experimental import pallas as pl
from jax.experimental.pallas import tpu as pltpu
from jax.experimental.pallas import tpu_sc as plsc

F32 = jnp.float32
NC = 2
NS = 16
NW = NC * NS


_GDN = lax.GatherDimensionNumbers(
    offset_dims=(), collapsed_slice_dims=(0,), start_index_map=(0,))


def _shfl(v, idx):
    return lax.gather(v, idx[:, None], _GDN, slice_sizes=(1,),
                      mode=lax.GatherScatterMode.PROMISE_IN_BOUNDS)


def _hsum_splat(v, lane):
    for sh in (8, 4, 2, 1):
        v = v + _shfl(v, lane ^ sh)
    return v


def _proj_body(x_ref, wl_ref, wr_ref, xl_ref, xr_ref):
    xb = x_ref[...]
    xl_ref[...] = jnp.dot(xb, wl_ref[...], preferred_element_type=F32)
    xr_ref[...] = jnp.dot(xb, wr_ref[...], preferred_element_type=F32)


def _proj(x, Wl, Wr, blk):
    n, d = x.shape
    grid = n // blk
    return pl.pallas_call(
        _proj_body,
        grid=(grid,),
        in_specs=[
            pl.BlockSpec((blk, d), lambda i: (i, 0)),
            pl.BlockSpec((d, d), lambda i: (0, 0)),
            pl.BlockSpec((d, d), lambda i: (0, 0)),
        ],
        out_specs=[
            pl.BlockSpec((blk, d), lambda i: (i, 0)),
            pl.BlockSpec((blk, d), lambda i: (i, 0)),
        ],
        out_shape=[
            jax.ShapeDtypeStruct((n, d), F32),
            jax.ShapeDtypeStruct((n, d), F32),
        ],
    )(x, Wl, Wr)


def _edge_pass_body(NB, B, NP, H, C,
                    xl_hbm, xr_hbm, src_hbm, dst_hbm, att_hbm, zu_hbm, zd_hbm,
                    u_out, den_out,
                    att_v, didx2d, sb0, sb1,
                    xlr0, xlr1, xrr0, xrr1, msg, exb,
                    u_acc, den_acc,
                    isem0, isem1, gsl0, gsl1, gsr0, gsr1, ssu, ssd):
    sbuf = (sb0, sb1)
    xlrs = (xlr0, xlr1)
    xrrs = (xrr0, xrr1)
    isem = (isem0, isem1)
    gsl = (gsl0, gsl1)
    gsr = (gsr0, gsr1)
    cid = lax.axis_index("c")
    sid = lax.axis_index("s")
    wid = sid * NC + cid
    D = H * C
    KV = D // 16
    EPW = NB * B

    pltpu.sync_copy(att_hbm, att_v)

    row0 = jnp.where(sid < 4, sid * 632, 2528 + (sid - 4) * 624)

    @pl.when(sid < 4)
    def _():
        pltpu.sync_copy(zu_hbm.at[pl.ds(row0, 632)],
                        u_acc.at[pl.ds(row0, 632)])
        pltpu.sync_copy(zd_hbm.at[pl.ds(row0, 632)],
                        den_acc.at[pl.ds(row0, 632)])

    @pl.when(sid >= 4)
    def _():
        pltpu.sync_copy(zu_hbm.at[pl.ds(row0, 624)],
                        u_acc.at[pl.ds(row0, 624)])
        pltpu.sync_copy(zd_hbm.at[pl.ds(row0, 624)],
                        den_acc.at[pl.ds(row0, 624)])

    plsc.subcore_barrier()

    attv = [att_v[pl.ds(16 * k, 16)] for k in range(KV)]
    lane = lax.broadcasted_iota(jnp.int32, (16,), 0)
    base = wid * EPW

    pltpu.sync_copy(dst_hbm.at[wid], didx2d)

    def compute(b):
        xlr, xrr = xlrs[b], xrrs[b]

        @plsc.parallel_loop(0, B, unroll=2)
        def _edge(e):
            a = [xlr[e, pl.ds(16 * k, 16)] for k in range(KV)]
            bb = [xrr[e, pl.ds(16 * k, 16)] for k in range(KV)]
            p = []
            for k in range(KV):
                sv = a[k] + bb[k]
                tv = jnp.maximum(sv, 0.2 * sv)
                p.append(tv * attv[k])
            exs = []
            for h in range(H):
                q = p[2 * h] + p[2 * h + 1]
                ev = jnp.exp(_hsum_splat(q, lane))
                exs.append(ev)
                msg[e, pl.ds(32 * h, 16)] = a[2 * h] * ev
                msg[e, pl.ds(32 * h + 16, 16)] = a[2 * h + 1] * ev
            exrow = jnp.where(
                lane == 0, exs[0],
                jnp.where(lane == 1, exs[1],
                          jnp.where(lane == 2, exs[2], exs[3])))
            exb[e, :] = exrow

    pltpu.sync_copy(src_hbm.at[pl.ds(base, B)], sbuf[0])
    pltpu.async_copy(xl_hbm.at[sbuf[0]], xlrs[0], gsl[0])
    pltpu.async_copy(xr_hbm.at[didx2d.at[0]], xrrs[0], gsr[0])
    pltpu.async_copy(src_hbm.at[pl.ds(base + B, B)], sbuf[1], isem[1])

    @pl.loop(0, NB, step=2)
    def _batch2(g0):
        for b in range(2):
            g = g0 + b
            bn = 1 - b
            pltpu.make_async_copy(xl_hbm.at[sbuf[b]], xlrs[b],
                                  gsl[b]).wait()
            pltpu.make_async_copy(xr_hbm.at[didx2d.at[g]], xrrs[b],
                                  gsr[b]).wait()

            @pl.when(g + 2 < NB)
            def _():
                pltpu.async_copy(
                    src_hbm.at[pl.ds(base + (g + 2) * B, B)],
                    sbuf[b], isem[b])

            @pl.when(g + 1 < NB)
            def _():
                pltpu.make_async_copy(
                    src_hbm.at[pl.ds(base + (g + 1) * B, B)],
                    sbuf[bn], isem[bn]).wait()
                pltpu.async_copy(xl_hbm.at[sbuf[bn]], xlrs[bn], gsl[bn])
                pltpu.async_copy(xr_hbm.at[didx2d.at[g + 1]], xrrs[bn],
                                 gsr[bn])

            @pl.when(g >= 1)
            def _():
                pltpu.make_async_copy(msg, u_acc.at[didx2d.at[g]],
                                      ssu).wait()
                pltpu.make_async_copy(exb, den_acc.at[didx2d.at[g]],
                                      ssd).wait()

            compute(b)
            pltpu.async_copy(msg, u_acc.at[didx2d.at[g]], ssu, add=True)
            pltpu.async_copy(exb, den_acc.at[didx2d.at[g]], ssd, add=True)

    pltpu.make_async_copy(msg, u_acc.at[didx2d.at[NB - 1]], ssu).wait()
    pltpu.make_async_copy(exb, den_acc.at[didx2d.at[NB - 1]], ssd).wait()

    plsc.subcore_barrier()

    @pl.when(sid < 4)
    def _():
        pltpu.sync_copy(u_acc.at[pl.ds(row0, 632)],
                        u_out.at[cid, pl.ds(row0, 632)])
        pltpu.sync_copy(den_acc.at[pl.ds(row0, 632)],
                        den_out.at[cid, pl.ds(row0, 632)])

    @pl.when(sid >= 4)
    def _():
        pltpu.sync_copy(u_acc.at[pl.ds(row0, 624)],
                        u_out.at[cid, pl.ds(row0, 624)])
        pltpu.sync_copy(den_acc.at[pl.ds(row0, 624)],
                        den_out.at[cid, pl.ds(row0, 624)])


def _edge_pass(xl, xr, src, dst, att_flat, H, C, NP):
    N, D = xl.shape
    E = src.shape[0]
    EPW = E // NW
    B = 40
    NB = EPW // B
    mesh = plsc.VectorSubcoreMesh(core_axis_name="c", subcore_axis_name="s")
    body = functools.partial(_edge_pass_body, NB, B, NP, H, C)
    f = pl.kernel(
        body,
        out_type=[
            jax.ShapeDtypeStruct((NC, NP, D), F32),
            jax.ShapeDtypeStruct((NC, NP, 16), F32),
        ],
        mesh=mesh,
        compiler_params=pltpu.CompilerParams(use_tc_tiling_on_sc=False),
        scratch_types=[
            pltpu.VMEM((D,), F32),
            pltpu.VMEM((NB, B), jnp.int32),
            pltpu.VMEM((B,), jnp.int32),
            pltpu.VMEM((B,), jnp.int32),
            pltpu.VMEM((B, D), F32),
            pltpu.VMEM((B, D), F32),
            pltpu.VMEM((B, D), F32),
            pltpu.VMEM((B, D), F32),
            pltpu.VMEM((B, D), F32),
            pltpu.VMEM((B, 16), F32),
            pltpu.VMEM_SHARED((NP, D), F32),
            pltpu.VMEM_SHARED((NP, 16), F32),
        ] + [pltpu.SemaphoreType.DMA] * 8,
    )
    zu = jnp.zeros((NP, D), F32)
    zd = jnp.zeros((NP, 16), F32)
    return f(xl, xr, src, dst.reshape(NW, NB, B), att_flat, zu, zd)


def _gin_agg_body(NB, B, NP,
                  x1_hbm, src_hbm, dst_hbm, zu_hbm, agg_out,
                  didx2d, sb0, sb1, rows0, rows1, acc,
                  isem0, isem1, gs0, gs1, ss0, ss1):
    cid = lax.axis_index("c")
    sid = lax.axis_index("s")
    wid = sid * NC + cid
    sbuf = (sb0, sb1)
    rows = (rows0, rows1)
    isem = (isem0, isem1)
    gs = (gs0, gs1)
    ss = (ss0, ss1)
    rows_per_tile = NP // NS
    EPW = NB * B
    base = wid * EPW

    row0 = sid * rows_per_tile
    pltpu.sync_copy(zu_hbm.at[pl.ds(row0, rows_per_tile)],
                    acc.at[pl.ds(row0, rows_per_tile)])
    plsc.subcore_barrier()

    pltpu.sync_copy(dst_hbm.at[wid], didx2d)

    pltpu.sync_copy(src_hbm.at[wid, 0], sbuf[0])
    pltpu.async_copy(x1_hbm.at[sbuf[0]], rows[0], gs[0])
    pltpu.async_copy(src_hbm.at[wid, 1], sbuf[1], isem[1])

    @pl.loop(0, NB, step=2)
    def _batch2(g0):
        for b in range(2):
            g = g0 + b
            bn = 1 - b
            pltpu.make_async_copy(x1_hbm.at[sbuf[b]], rows[b],
                                  gs[b]).wait()

            @pl.when(g + 2 < NB)
            def _():
                pltpu.async_copy(src_hbm.at[wid, g + 2], sbuf[b], isem[b])

            @pl.when(g + 1 < NB)
            def _():
                @pl.when(g >= 1)
                def _():
                    pltpu.make_async_copy(rows[bn],
                                          acc.at[didx2d.at[g]],
                                          ss[bn]).wait()

                pltpu.make_async_copy(src_hbm.at[wid, g + 1],
                                      sbuf[bn], isem[bn]).wait()
                pltpu.async_copy(x1_hbm.at[sbuf[bn]], rows[bn], gs[bn])

            pltpu.async_copy(rows[b], acc.at[didx2d.at[g]], ss[b],
                             add=True)

    for b in range(2):
        pltpu.make_async_copy(rows[b], acc.at[didx2d.at[b]], ss[b]).wait()

    plsc.subcore_barrier()
    pltpu.sync_copy(acc.at[pl.ds(row0, rows_per_tile)],
                    agg_out.at[cid, pl.ds(row0, rows_per_tile)])


def _gin_agg(x1, src, dst, NP):
    N, D = x1.shape
    E = src.shape[0]
    EPW = E // NW
    B = 125
    NB = EPW // B
    mesh = plsc.VectorSubcoreMesh(core_axis_name="c", subcore_axis_name="s")
    body = functools.partial(_gin_agg_body, NB, B, NP)
    f = pl.kernel(
        body,
        out_type=jax.ShapeDtypeStruct((NC, NP, D), F32),
        mesh=mesh,
        compiler_params=pltpu.CompilerParams(use_tc_tiling_on_sc=False),
        scratch_types=[
            pltpu.VMEM((NB, B), jnp.int32),
            pltpu.VMEM((B,), jnp.int32),
            pltpu.VMEM((B,), jnp.int32),
            pltpu.VMEM((B, D), F32),
            pltpu.VMEM((B, D), F32),
            pltpu.VMEM_SHARED((NP, D), F32),
        ] + [pltpu.SemaphoreType.DMA] * 6,
    )
    zu = jnp.zeros((NP, D), F32)
    return f(x1, src.reshape(NW, NB, B), dst.reshape(NW, NB, B), zu)


def _finalize_body(xl_ref, xr_ref, u0_ref, u1_ref, d0_ref, d1_ref,
                   att_ref, mh_ref, md_ref, bias_ref, x1_ref):
    xl = xl_ref[...]
    blk, d = xl.shape
    s = xl + xr_ref[...]
    t = jnp.maximum(s, 0.2 * s) * att_ref[...]
    sfull = jnp.dot(t, mh_ref[...], preferred_element_type=F32)
    exs = jnp.exp(sfull)
    den = (d0_ref[...] + d1_ref[...]).reshape(blk, 16)
    denf = jnp.dot(den, md_ref[...], preferred_element_type=F32)
    num = (u0_ref[...] + u1_ref[...]).reshape(blk, d) + exs * xl
    gat = num / (denf + exs + 1e-16) + bias_ref[...]
    x1_ref[...] = jnp.maximum(gat, 0.01 * gat)


def _finalize(xl, xr, u, den, att_row, mhead, mden, bias_row, blk):
    n, d = xl.shape
    grid = n // blk
    nspec = pl.BlockSpec((blk, d), lambda i: (i, 0))
    wspec0 = pl.BlockSpec((1, d), lambda i: (0, 0))
    return pl.pallas_call(
        _finalize_body,
        grid=(grid,),
        in_specs=[
            nspec, nspec,
            pl.BlockSpec((1, blk, d), lambda i: (0, i, 0)),
            pl.BlockSpec((1, blk, d), lambda i: (1, i, 0)),
            pl.BlockSpec((1, blk, 16), lambda i: (0, i, 0)),
            pl.BlockSpec((1, blk, 16), lambda i: (1, i, 0)),
            wspec0,
            pl.BlockSpec((d, d), lambda i: (0, 0)),
            pl.BlockSpec((16, d), lambda i: (0, 0)),
            wspec0,
        ],
        out_specs=nspec,
        out_shape=jax.ShapeDtypeStruct((n, d), F32),
    )(xl, xr, u, u, den, den, att_row, mhead, mden, bias_row)


def _mlp_pool_body(G, x1_ref, a0_ref, a1_ref, bt_ref, w1_ref, b1_ref,
                   w2_ref, b2_ref, out_ref):
    i = pl.program_id(0)
    x1 = x1_ref[...]
    h0 = x1 + (a0_ref[...] + a1_ref[...]).reshape(x1.shape)
    h1 = jnp.maximum(jnp.dot(h0, w1_ref[...], preferred_element_type=F32)
                     + b1_ref[...], 0.0)
    h2 = jnp.maximum(jnp.dot(h1, w2_ref[...], preferred_element_type=F32)
                     + b2_ref[...], 0.0)
    blk = h2.shape[0]
    gids = lax.broadcasted_iota(jnp.int32, (G, blk), 0)
    bt = bt_ref[...].reshape(1, blk)
    onehot = jnp.where(gids == jnp.broadcast_to(bt, (G, blk)),
                       1.0, 0.0).astype(F32)
    pp = jnp.dot(onehot, h2, preferred_element_type=F32)

    @pl.when(i == 0)
    def _():
        out_ref[...] = jnp.zeros_like(out_ref)

    out_ref[...] += pp


def _mlp_pool(x1, agg, bt, w1, b1_row, w2, b2_row, G, blk):
    n, d = x1.shape
    grid = n // blk
    nspec = pl.BlockSpec((blk, d), lambda i: (i, 0))
    wspec = pl.BlockSpec((d, d), lambda i: (0, 0))
    bspec = pl.BlockSpec((1, d), lambda i: (0, 0))
    return pl.pallas_call(
        functools.partial(_mlp_pool_body, G),
        grid=(grid,),
        in_specs=[
            nspec,
            pl.BlockSpec((1, blk, d), lambda i: (0, i, 0)),
            pl.BlockSpec((1, blk, d), lambda i: (1, i, 0)),
            pl.BlockSpec((1, 1, blk), lambda i: (i, 0, 0)),
            wspec, bspec, wspec, bspec,
        ],
        out_specs=pl.BlockSpec((G, d), lambda i: (0, 0)),
        out_shape=jax.ShapeDtypeStruct((G, d), F32),
    )(x1, agg, agg, bt, w1, b1_row, w2, b2_row)


@jax.jit
def _run(x, edge_index, batch, Wl, Wr, att, gat_bias, gin_w1, gin_b1,
         bn_gamma, bn_beta, gin_w2, gin_b2):
    N, D = x.shape
    H, C = att.shape
    G = 64
    blk = 1000

    src = edge_index[0]
    dst = edge_index[1]

    att_flat = att.reshape(D)
    att_row = att_flat.reshape(1, D)
    ci = jnp.arange(D, dtype=jnp.int32) // C
    mhead = (ci[:, None] == ci[None, :]).astype(F32)
    mden = (jnp.arange(16, dtype=jnp.int32)[:, None] == ci[None, :]).astype(F32)
    bias_row = gat_bias.reshape(1, D)
    bn_s = bn_gamma / math.sqrt(1.0 + 1e-5)
    w1p = gin_w1 * bn_s[None, :]
    b1_row = (gin_b1 * bn_s + bn_beta).reshape(1, D)
    b2_row = gin_b2.reshape(1, D)
    bt = batch.reshape(N // blk, 1, blk)

    NPE = 10016
    NP = 10240
    xl, xr = _proj(x, Wl, Wr, blk)
    u, den = _edge_pass(xl, xr, src, dst, att_flat, H, C, NPE)
    x1 = _finalize(xl, xr, u, den, att_row, mhead, mden, bias_row, blk)
    agg = _gin_agg(x1, src, dst, NP)
    return _mlp_pool(x1, agg, bt, w1p, b1_row, gin_w2, b2_row, G, blk)


def kernel(x, edge_index, batch, Wl, Wr, att, gat_bias, gin_w1, gin_b1,
           bn_gamma, bn_beta, gin_w2, gin_b2):
    return _run(x, edge_index, batch, Wl, Wr, att, gat_bias, gin_w1, gin_b1,
                bn_gamma, bn_beta, gin_w2, gin_b2)
